# baseline (speedup 1.0000x reference)
import functools

import jax
import jax.numpy as jnp
from jax import lax
from jax.experimental import pallas as pl
from jax.experimental.pallas import tpu as pltpu
from jax.experimental.pallas import tpu_sc as plsc

N = 10000
E = 160000
DIN = 256
DOUT = 256
H = 128
NEG_SLOPE = 0.2

NC = 2
NS = 16
CHK = 128
NR = 80
EPS = NR * CHK
EP = NS * EPS
PADN = EP - E
NP = 10112
RPT = NP // NS

_MESH = plsc.VectorSubcoreMesh(
    core_axis_name="c", subcore_axis_name="s", num_cores=NC, num_subcores=NS)


def _tc_body(h_ref, w_ref, a_ref, tbl_ref, sc_ref):
    hb = h_ref[...]
    f0 = jnp.dot(hb, w_ref[0], preferred_element_type=jnp.float32)
    f1 = jnp.dot(hb, w_ref[1], preferred_element_type=jnp.float32)
    tbl_ref[0] = f0[:, :H]
    tbl_ref[1] = f0[:, H:]
    tbl_ref[2] = f1[:, :H]
    tbl_ref[3] = f1[:, H:]
    el0 = jnp.dot(f0, a_ref[0, 0], preferred_element_type=jnp.float32)
    er0 = jnp.dot(f0, a_ref[0, 1], preferred_element_type=jnp.float32)
    el1 = jnp.dot(f1, a_ref[1, 0], preferred_element_type=jnp.float32)
    er1 = jnp.dot(f1, a_ref[1, 1], preferred_element_type=jnp.float32)
    sc_ref[...] = jnp.stack([el0, er0, el1, er1], axis=1)


_BN = 1000


def _tc_feat(h, Ws, As):
    return pl.pallas_call(
        _tc_body,
        grid=(N // _BN,),
        in_specs=[
            pl.BlockSpec((_BN, DIN), lambda i: (i, 0)),
            pl.BlockSpec((2, DIN, DOUT), lambda i: (0, 0, 0)),
            pl.BlockSpec((2, 2, DOUT), lambda i: (0, 0, 0)),
        ],
        out_specs=[
            pl.BlockSpec((4, _BN, H), lambda i: (0, i, 0)),
            pl.BlockSpec((_BN, 4), lambda i: (i, 0)),
        ],
        out_shape=[
            jax.ShapeDtypeStruct((4, N, H), jnp.float32),
            jax.ShapeDtypeStruct((N, 4), jnp.float32),
        ],
    )(h, Ws, As)


def _ka_body(el0, er0, el1, er1, srcp, dstp, alpha_out,
             denom_sp, el_v, er_v, src_v, dst_v, ee_v, denom_v, dsem):
    c = lax.axis_index("c")
    s = lax.axis_index("s")

    @pl.when(s == 0)
    def _zero():
        def zb(i, _):
            denom_v[pl.ds(i * 16, 16)] = jnp.zeros((16,), jnp.float32)
            return 0
        lax.fori_loop(0, N // 16, zb, 0)
        pltpu.sync_copy(denom_v, denom_sp)

    @pl.when(c == 0)
    def _load0():
        pltpu.sync_copy(el0, el_v)
        pltpu.sync_copy(er0, er_v)

    @pl.when(c == 1)
    def _load1():
        pltpu.sync_copy(el1, el_v)
        pltpu.sync_copy(er1, er_v)
    pltpu.sync_copy(srcp.at[c, s], src_v)
    pltpu.sync_copy(dstp.at[c, s], dst_v)
    plsc.subcore_barrier()

    base = s * EPS

    def row(j, _):
        for k in range(CHK // 16):
            srcv = src_v[j, pl.ds(k * 16, 16)]
            dstv = dst_v[j, pl.ds(k * 16, 16)]
            e = plsc.load_gather(el_v, [srcv]) + plsc.load_gather(er_v, [dstv])
            e = jnp.where(e >= 0.0, e, e * NEG_SLOPE)
            ee = jnp.exp(e)
            eid = base + j * CHK + k * 16 + lax.iota(jnp.int32, 16)
            ee = jnp.where(eid < E, ee, 0.0)
            ee_v[j, pl.ds(k * 16, 16)] = ee
        pltpu.async_copy(ee_v.at[j], denom_sp.at[dst_v.at[j]], dsem,
                         add=True)

        @pl.when(j >= 16)
        def _lagged_drain():
            pltpu.make_async_copy(ee_v.at[0], denom_sp.at[dst_v.at[0]],
                                  dsem).wait()
        return 0

    lax.fori_loop(0, NR, row, 0)

    def drain(j, _):
        pltpu.make_async_copy(ee_v.at[0], denom_sp.at[dst_v.at[0]],
                              dsem).wait()
        return 0

    lax.fori_loop(0, 16, drain, 0)
    plsc.subcore_barrier()
    pltpu.sync_copy(denom_sp, denom_v)

    def row2(j, _):
        for k in range(CHK // 16):
            dstv = dst_v[j, pl.ds(k * 16, 16)]
            dg = plsc.load_gather(denom_v, [dstv])
            a = ee_v[j, pl.ds(k * 16, 16)] / (dg + 1e-9) * 0.5
            ee_v[j, pl.ds(k * 16, 16)] = a
        return 0

    lax.fori_loop(0, NR, row2, 0)
    pltpu.sync_copy(ee_v, alpha_out.at[c, s])


_SC_PARAMS = pltpu.CompilerParams(needs_layout_passes=False)

_ka = functools.partial(
    pl.kernel,
    out_type=jax.ShapeDtypeStruct((NC, NS, NR, CHK), jnp.float32),
    mesh=_MESH,
    compiler_params=_SC_PARAMS,
    scratch_types=[
        pltpu.VMEM_SHARED((N,), jnp.float32),
        pltpu.VMEM((N,), jnp.float32),
        pltpu.VMEM((N,), jnp.float32),
        pltpu.VMEM((NR, CHK), jnp.int32),
        pltpu.VMEM((NR, CHK), jnp.int32),
        pltpu.VMEM((NR, CHK), jnp.float32),
        pltpu.VMEM((N,), jnp.float32),
        pltpu.SemaphoreType.DMA,
    ],
)(_ka_body)


_NB = 3
_TAIL = N - (NS - 1) * RPT - 4 * CHK


def _kb_body(tbl4, srcp, dstp, alphap, outh,
             accum_sp,
             b0, b1, b2, sr0, sr1, sr2, dr0, dr1, dr2, ar0, ar1, ar2,
             gs0, gs1, gs2, ss0, ss1, ss2, is0, is1, is2):
    c = lax.axis_index("c")
    s = lax.axis_index("s")
    bufs = [b0, b1, b2]
    srs = [sr0, sr1, sr2]
    drs = [dr0, dr1, dr2]
    ars = [ar0, ar1, ar2]
    gsems = [gs0, gs1, gs2]
    ssems = [ss0, ss1, ss2]
    isems = [is0, is1, is2]

    def zb(i, _):
        for u in range(H // 16):
            b0[i, pl.ds(u * 16, 16)] = jnp.zeros((16,), jnp.float32)
        return 0
    lax.fori_loop(0, CHK, zb, 0)
    row0 = s * RPT
    for i in range(4):
        pltpu.sync_copy(b0, accum_sp.at[pl.ds(row0 + i * CHK, CHK)])

    @pl.when(s < NS - 1)
    def _z5():
        pltpu.sync_copy(b0.at[pl.ds(0, RPT - 4 * CHK)],
                        accum_sp.at[pl.ds(row0 + 4 * CHK, RPT - 4 * CHK)])

    @pl.when(s == NS - 1)
    def _z5t():
        pltpu.sync_copy(b0.at[pl.ds(0, _TAIL)],
                        accum_sp.at[pl.ds(row0 + 4 * CHK, _TAIL)])

    plsc.subcore_barrier()

    for m in range(2):
        off = (2 * m) * N + c * N

        def idx_issue(jj, b):
            pltpu.async_copy(srcp.at[m, s, jj], srs[b], isems[b])
            pltpu.async_copy(dstp.at[m, s, jj], drs[b], isems[b])
            pltpu.async_copy(alphap.at[m, s, jj], ars[b], isems[b])

        def idx_wait(b):
            pltpu.make_async_copy(srcp.at[m, s, 0], srs[b], isems[b]).wait()
            pltpu.make_async_copy(dstp.at[m, s, 0], drs[b], isems[b]).wait()
            pltpu.make_async_copy(alphap.at[m, s, 0], ars[b], isems[b]).wait()

        def gather_issue(b):
            for u in range(CHK // 16):
                srs[b][pl.ds(u * 16, 16)] = srs[b][pl.ds(u * 16, 16)] + off
            pltpu.async_copy(tbl4.at[srs[b]], bufs[b], gsems[b])

        def gather_wait(b):
            pltpu.make_async_copy(tbl4.at[srs[b]], bufs[b], gsems[b]).wait()

        def do_scale(b):
            def body(k, _):
                ab = plsc.load_gather(ars[b], [jnp.full((16,), k, jnp.int32)])
                for u in range(H // 16):
                    bufs[b][k, pl.ds(u * 16, 16)] = (
                        bufs[b][k, pl.ds(u * 16, 16)] * ab)
                return 0
            lax.fori_loop(0, CHK, body, 0, unroll=8)

        def scatter_issue(b):
            pltpu.async_copy(bufs[b], accum_sp.at[drs[b]], ssems[b], add=True)

        def scatter_wait(b):
            pltpu.make_async_copy(bufs[b], accum_sp.at[drs[b]],
                                  ssems[b]).wait()

        for b in range(_NB):
            idx_issue(b, b)
        for b in range(_NB):
            idx_wait(b)
            gather_issue(b)

        def outer(g, _):
            for b in range(_NB):
                jj = _NB * g + b

                @pl.when(jj < NR)
                def _work(b=b, jj=jj):
                    gather_wait(b)
                    do_scale(b)
                    scatter_issue(b)

                @pl.when(jj + _NB < NR)
                def _recycle(b=b, jj=jj):
                    scatter_wait(b)
                    idx_issue(jj + _NB, b)
                    idx_wait(b)
                    gather_issue(b)
            return 0

        lax.fori_loop(0, NR // _NB + 1, outer, 0)
        for b in range(_NB):
            scatter_wait(b)

    plsc.subcore_barrier()
    for i in range(4):
        pltpu.sync_copy(accum_sp.at[pl.ds(row0 + i * CHK, CHK)],
                        outh.at[pl.ds(row0 + i * CHK, CHK), pl.ds(c * H, H)])

    @pl.when(s < NS - 1)
    def _w5():
        pltpu.sync_copy(
            accum_sp.at[pl.ds(row0 + 4 * CHK, RPT - 4 * CHK)],
            outh.at[pl.ds(row0 + 4 * CHK, RPT - 4 * CHK), pl.ds(c * H, H)])

    @pl.when(s == NS - 1)
    def _w5t():
        pltpu.sync_copy(
            accum_sp.at[pl.ds(row0 + 4 * CHK, _TAIL)],
            outh.at[pl.ds(row0 + 4 * CHK, _TAIL), pl.ds(c * H, H)])


_kb = functools.partial(
    pl.kernel,
    out_type=jax.ShapeDtypeStruct((N, DOUT), jnp.float32),
    mesh=_MESH,
    compiler_params=_SC_PARAMS,
    scratch_types=(
        [pltpu.VMEM_SHARED((N, H), jnp.float32)]
        + [pltpu.VMEM((CHK, H), jnp.float32)] * _NB
        + [pltpu.VMEM((CHK,), jnp.int32)] * _NB
        + [pltpu.VMEM((CHK,), jnp.int32)] * _NB
        + [pltpu.VMEM((CHK,), jnp.float32)] * _NB
        + [pltpu.SemaphoreType.DMA] * (3 * _NB)
    ),
)(_kb_body)


def kernel(h, edge_index0, edge_index1, W0, al0, ar0, W1, al1, ar1):
    Ws = jnp.stack([W0, W1])
    As = jnp.stack([jnp.stack([al0, ar0]), jnp.stack([al1, ar1])])
    tbl, sc = _tc_feat(h, Ws, As)

    pad = (jnp.arange(PADN, dtype=jnp.int32) * 97) % N

    def prep(ei):
        sfull = jnp.concatenate([ei[0], pad]).reshape(NS, NR, CHK)
        dfull = jnp.concatenate([ei[1], pad]).reshape(NS, NR, CHK)
        return sfull, dfull

    s0, d0 = prep(edge_index0)
    s1, d1 = prep(edge_index1)
    srcp = jnp.stack([s0, s1])
    dstp = jnp.stack([d0, d1])

    alpha = _ka(sc[:, 0], sc[:, 1], sc[:, 2], sc[:, 3], srcp, dstp)
    return _kb(tbl.reshape(4 * N, H), srcp, dstp, alpha)


# speedup vs baseline: 1.1777x; 1.1777x over previous
import functools

import jax
import jax.numpy as jnp
from jax import lax
from jax.experimental import pallas as pl
from jax.experimental.pallas import tpu as pltpu
from jax.experimental.pallas import tpu_sc as plsc

N = 10000
E = 160000
DIN = 256
DOUT = 256
H = 128
NEG_SLOPE = 0.2

NC = 2
NS = 16
CW = 64
NRW = 160
EPS = NRW * CW
EP = NS * EPS
PADN = EP - E
NP = 10112
RPT = NP // NS
_TAIL = N - (NS - 1) * RPT

_MESH = plsc.VectorSubcoreMesh(
    core_axis_name="c", subcore_axis_name="s", num_cores=NC, num_subcores=NS)
_SC_PARAMS = pltpu.CompilerParams(needs_layout_passes=False)


def _tc_body(h_ref, w_ref, a_ref, tbl_ref, sc_ref):
    hb = h_ref[...]
    f0 = jnp.dot(hb, w_ref[0], preferred_element_type=jnp.float32)
    f1 = jnp.dot(hb, w_ref[1], preferred_element_type=jnp.float32)
    tbl_ref[0] = f0[:, :H]
    tbl_ref[1] = f0[:, H:]
    tbl_ref[2] = f1[:, :H]
    tbl_ref[3] = f1[:, H:]
    el0 = jnp.dot(f0, a_ref[0, 0], preferred_element_type=jnp.float32)
    er0 = jnp.dot(f0, a_ref[0, 1], preferred_element_type=jnp.float32)
    el1 = jnp.dot(f1, a_ref[1, 0], preferred_element_type=jnp.float32)
    er1 = jnp.dot(f1, a_ref[1, 1], preferred_element_type=jnp.float32)
    sc_ref[...] = jnp.stack([el0, er0, el1, er1], axis=1)


_BN = 1000


def _tc_feat(h, Ws, As):
    return pl.pallas_call(
        _tc_body,
        grid=(N // _BN,),
        in_specs=[
            pl.BlockSpec((_BN, DIN), lambda i: (i, 0)),
            pl.BlockSpec((2, DIN, DOUT), lambda i: (0, 0, 0)),
            pl.BlockSpec((2, 2, DOUT), lambda i: (0, 0, 0)),
        ],
        out_specs=[
            pl.BlockSpec((4, _BN, H), lambda i: (0, i, 0)),
            pl.BlockSpec((_BN, 4), lambda i: (i, 0)),
        ],
        out_shape=[
            jax.ShapeDtypeStruct((4, N, H), jnp.float32),
            jax.ShapeDtypeStruct((N, 4), jnp.float32),
        ],
    )(h, Ws, As)


def _ka_body(el0, er0, el1, er1, srcp, dstp, alpha_out,
             denom_sp, el_v, er_v, src_v, dst_v, ee_v, denom_v, dsem):
    c = lax.axis_index("c")
    s = lax.axis_index("s")

    @pl.when(s == 0)
    def _zero():
        def zb(i, _):
            denom_v[pl.ds(i * 16, 16)] = jnp.zeros((16,), jnp.float32)
            return 0
        lax.fori_loop(0, N // 16, zb, 0)
        pltpu.sync_copy(denom_v, denom_sp)

    @pl.when(c == 0)
    def _load0():
        pltpu.sync_copy(el0, el_v)
        pltpu.sync_copy(er0, er_v)

    @pl.when(c == 1)
    def _load1():
        pltpu.sync_copy(el1, el_v)
        pltpu.sync_copy(er1, er_v)
    pltpu.sync_copy(srcp.at[c, s], src_v)
    pltpu.sync_copy(dstp.at[c, s], dst_v)
    plsc.subcore_barrier()

    base = s * EPS

    def row(j, _):
        for k in range(CW // 16):
            srcv = src_v[j, pl.ds(k * 16, 16)]
            dstv = dst_v[j, pl.ds(k * 16, 16)]
            e = plsc.load_gather(el_v, [srcv]) + plsc.load_gather(er_v, [dstv])
            e = jnp.where(e >= 0.0, e, e * NEG_SLOPE)
            ee = jnp.exp(e)
            eid = base + j * CW + k * 16 + lax.iota(jnp.int32, 16)
            ee = jnp.where(eid < E, ee, 0.0)
            ee_v[j, pl.ds(k * 16, 16)] = ee
        pltpu.async_copy(ee_v.at[j], denom_sp.at[dst_v.at[j]], dsem,
                         add=True)

        @pl.when(j >= 16)
        def _lagged_drain():
            pltpu.make_async_copy(ee_v.at[0], denom_sp.at[dst_v.at[0]],
                                  dsem).wait()
        return 0

    lax.fori_loop(0, NRW, row, 0)

    def drain(j, _):
        pltpu.make_async_copy(ee_v.at[0], denom_sp.at[dst_v.at[0]],
                              dsem).wait()
        return 0

    lax.fori_loop(0, 16, drain, 0)
    plsc.subcore_barrier()
    pltpu.sync_copy(denom_sp, denom_v)

    def row2(j, _):
        for k in range(CW // 16):
            dstv = dst_v[j, pl.ds(k * 16, 16)]
            dg = plsc.load_gather(denom_v, [dstv])
            a = ee_v[j, pl.ds(k * 16, 16)] / (dg + 1e-9) * 0.5
            ee_v[j, pl.ds(k * 16, 16)] = a
        return 0

    lax.fori_loop(0, NRW, row2, 0)
    pltpu.sync_copy(ee_v, alpha_out.at[c, s])


_ka = functools.partial(
    pl.kernel,
    out_type=jax.ShapeDtypeStruct((NC, NS, NRW, CW), jnp.float32),
    mesh=_MESH,
    compiler_params=_SC_PARAMS,
    scratch_types=[
        pltpu.VMEM_SHARED((N,), jnp.float32),
        pltpu.VMEM((N,), jnp.float32),
        pltpu.VMEM((N,), jnp.float32),
        pltpu.VMEM((NRW, CW), jnp.int32),
        pltpu.VMEM((NRW, CW), jnp.int32),
        pltpu.VMEM((NRW, CW), jnp.float32),
        pltpu.VMEM((N,), jnp.float32),
        pltpu.SemaphoreType.DMA,
    ],
)(_ka_body)


_NB = 4
_NQ = 4
_NH = NRW // _NQ


def _kb_body(tbl4, srcp, dstp, alphap, outh,
             accum_sp, src_v, dst_v, alpha_v,
             b0, b1, b2, b3, gs0, gs1, gs2, gs3, ss0, ss1, ss2, ss3):
    c = lax.axis_index("c")
    s = lax.axis_index("s")
    bufs = [b0, b1, b2, b3]
    gsems = [gs0, gs1, gs2, gs3]
    ssems = [ss0, ss1, ss2, ss3]

    def zb(i, _):
        for u in range(H // 16):
            b0[i, pl.ds(u * 16, 16)] = jnp.zeros((16,), jnp.float32)
        return 0
    lax.fori_loop(0, CW, zb, 0)
    row0 = s * RPT

    def stage_rows(write_out):
        def one(lo, sz):
            if write_out:
                pltpu.sync_copy(
                    accum_sp.at[pl.ds(row0 + lo, sz)],
                    outh.at[pl.ds(row0 + lo, sz), pl.ds(c * H, H)])
            else:
                pltpu.sync_copy(b0.at[pl.ds(0, sz)],
                                accum_sp.at[pl.ds(row0 + lo, sz)])
        for i in range(8):
            one(i * CW, CW)

        @pl.when(s < NS - 1)
        def _full():
            one(8 * CW, CW)
            one(9 * CW, RPT - 9 * CW)

        @pl.when(s == NS - 1)
        def _tail():
            one(8 * CW, _TAIL - 8 * CW)

    stage_rows(False)
    plsc.subcore_barrier()

    def seg_loop(seg, _):
        m = seg // _NQ
        hh = seg % _NQ
        off = 2 * N * m + c * N
        q0 = pl.multiple_of(hh * _NH, 8)
        pltpu.sync_copy(srcp.at[m, s, pl.ds(q0, _NH)], src_v)
        pltpu.sync_copy(dstp.at[m, s, pl.ds(q0, _NH)], dst_v)
        pltpu.sync_copy(alphap.at[m, s, pl.ds(q0, _NH)], alpha_v)

        def gather_issue(jj, b):
            for u in range(CW // 16):
                src_v[jj, pl.ds(u * 16, 16)] = (
                    src_v[jj, pl.ds(u * 16, 16)] + off)
            pltpu.async_copy(tbl4.at[src_v.at[jj]], bufs[b], gsems[b])

        def gather_wait(jj, b):
            pltpu.make_async_copy(
                tbl4.at[src_v.at[jj]], bufs[b], gsems[b]).wait()

        def do_scale(jj, b):
            rv = jnp.full((16,), jj, jnp.int32)

            def body(k, _):
                ab = plsc.load_gather(
                    alpha_v, [rv, jnp.full((16,), k, jnp.int32)])
                for u in range(H // 16):
                    bufs[b][k, pl.ds(u * 16, 16)] = (
                        bufs[b][k, pl.ds(u * 16, 16)] * ab)
                return 0
            lax.fori_loop(0, CW, body, 0, unroll=8)

        def scatter_issue(jj, b):
            pltpu.async_copy(bufs[b], accum_sp.at[dst_v.at[jj]],
                             ssems[b], add=True)

        def scatter_wait(jj, b):
            pltpu.make_async_copy(bufs[b], accum_sp.at[dst_v.at[jj]],
                                  ssems[b]).wait()

        for b in range(_NB):
            gather_issue(b, b)

        def outer(g, _):
            for b in range(_NB):
                jj = _NB * g + b
                gather_wait(jj, b)
                do_scale(jj, b)
                scatter_issue(jj, b)
                tt = jj + 2
                bb = (b + 2) % _NB

                @pl.when(jnp.logical_and(tt >= _NB, tt < _NH))
                def _recycle(tt=tt, bb=bb):
                    scatter_wait(tt - _NB, bb)
                    gather_issue(tt, bb)
            return 0

        lax.fori_loop(0, _NH // _NB, outer, 0)
        for b in range(_NB):
            scatter_wait(_NH - _NB + b, b)
        return 0

    lax.fori_loop(0, 2 * _NQ, seg_loop, 0)

    plsc.subcore_barrier()
    stage_rows(True)


_kb = functools.partial(
    pl.kernel,
    out_type=jax.ShapeDtypeStruct((N, DOUT), jnp.float32),
    mesh=_MESH,
    compiler_params=_SC_PARAMS,
    scratch_types=(
        [pltpu.VMEM_SHARED((N, H), jnp.float32),
         pltpu.VMEM((_NH, CW), jnp.int32),
         pltpu.VMEM((_NH, CW), jnp.int32),
         pltpu.VMEM((_NH, CW), jnp.float32)]
        + [pltpu.VMEM((CW, H), jnp.float32)] * _NB
        + [pltpu.SemaphoreType.DMA] * (2 * _NB)
    ),
)(_kb_body)


def kernel(h, edge_index0, edge_index1, W0, al0, ar0, W1, al1, ar1):
    Ws = jnp.stack([W0, W1])
    As = jnp.stack([jnp.stack([al0, ar0]), jnp.stack([al1, ar1])])
    tbl, sc = _tc_feat(h, Ws, As)

    pad = (jnp.arange(PADN, dtype=jnp.int32) * 97) % N

    def prep(ei):
        sfull = jnp.concatenate([ei[0], pad]).reshape(NS, NRW, CW)
        dfull = jnp.concatenate([ei[1], pad]).reshape(NS, NRW, CW)
        return sfull, dfull

    s0, d0 = prep(edge_index0)
    s1, d1 = prep(edge_index1)
    srcp = jnp.stack([s0, s1])
    dstp = jnp.stack([d0, d1])

    alpha = _ka(sc[:, 0], sc[:, 1], sc[:, 2], sc[:, 3], srcp, dstp)
    return _kb(tbl.reshape(4 * N, H), srcp, dstp, alpha)


# speedup vs baseline: 1.1780x; 1.0003x over previous
import functools

import jax
import jax.numpy as jnp
from jax import lax
from jax.experimental import pallas as pl
from jax.experimental.pallas import tpu as pltpu
from jax.experimental.pallas import tpu_sc as plsc

N = 10000
E = 160000
DIN = 256
DOUT = 256
H = 128
NEG_SLOPE = 0.2

NC = 2
NS = 16
CW = 64
NRW = 160
EPS = NRW * CW
EP = NS * EPS
PADN = EP - E
NP = 10112
RPT = NP // NS
_TAIL = N - (NS - 1) * RPT

_MESH = plsc.VectorSubcoreMesh(
    core_axis_name="c", subcore_axis_name="s", num_cores=NC, num_subcores=NS)
_SC_PARAMS = pltpu.CompilerParams(needs_layout_passes=False)


_BN = 1000


def _sc_body(h_ref, w_ref, a_ref, sc_ref):
    hb = h_ref[...]
    cols = []
    for m in range(2):
        for lr in range(2):
            wa = jnp.dot(w_ref[m], a_ref[m, lr],
                         preferred_element_type=jnp.float32)
            cols.append(jnp.dot(hb, wa, preferred_element_type=jnp.float32))
    sc_ref[...] = jnp.stack(cols, axis=1)


def _tc_scores(h, Ws, As):
    return pl.pallas_call(
        _sc_body,
        grid=(N // _BN,),
        in_specs=[
            pl.BlockSpec((_BN, DIN), lambda i: (i, 0)),
            pl.BlockSpec((2, DIN, DOUT), lambda i: (0, 0, 0)),
            pl.BlockSpec((2, 2, DOUT), lambda i: (0, 0, 0)),
        ],
        out_specs=pl.BlockSpec((_BN, 4), lambda i: (i, 0)),
        out_shape=jax.ShapeDtypeStruct((N, 4), jnp.float32),
    )(h, Ws, As)


def _tbl_body(h_ref, w_ref, tbl_ref):
    tbl_ref[...] = jnp.dot(h_ref[...], w_ref[0],
                           preferred_element_type=jnp.float32)


def _tc_tbl(h, Wh):
    return pl.pallas_call(
        _tbl_body,
        grid=(N // _BN, 4),
        in_specs=[
            pl.BlockSpec((_BN, DIN), lambda i, q: (i, 0)),
            pl.BlockSpec((1, DIN, H), lambda i, q: (q, 0, 0)),
        ],
        out_specs=pl.BlockSpec((_BN, H), lambda i, q: (q * (N // _BN) + i, 0)),
        out_shape=jax.ShapeDtypeStruct((4 * N, H), jnp.float32),
    )(h, Wh)


def _ka_body(el0, er0, el1, er1, srcp, dstp, alpha_out,
             denom_sp, el_v, er_v, src_v, dst_v, ee_v, denom_v, dsem):
    c = lax.axis_index("c")
    s = lax.axis_index("s")

    @pl.when(s == 0)
    def _zero():
        def zb(i, _):
            denom_v[pl.ds(i * 16, 16)] = jnp.zeros((16,), jnp.float32)
            return 0
        lax.fori_loop(0, N // 16, zb, 0)
        pltpu.sync_copy(denom_v, denom_sp)

    @pl.when(c == 0)
    def _load0():
        pltpu.sync_copy(el0, el_v)
        pltpu.sync_copy(er0, er_v)

    @pl.when(c == 1)
    def _load1():
        pltpu.sync_copy(el1, el_v)
        pltpu.sync_copy(er1, er_v)
    pltpu.sync_copy(srcp.at[c, s], src_v)
    pltpu.sync_copy(dstp.at[c, s], dst_v)
    plsc.subcore_barrier()

    base = s * EPS

    def row(j, _):
        for k in range(CW // 16):
            srcv = src_v[j, pl.ds(k * 16, 16)]
            dstv = dst_v[j, pl.ds(k * 16, 16)]
            e = plsc.load_gather(el_v, [srcv]) + plsc.load_gather(er_v, [dstv])
            e = jnp.where(e >= 0.0, e, e * NEG_SLOPE)
            ee = jnp.exp(e)
            eid = base + j * CW + k * 16 + lax.iota(jnp.int32, 16)
            ee = jnp.where(eid < E, ee, 0.0)
            ee_v[j, pl.ds(k * 16, 16)] = ee
        pltpu.async_copy(ee_v.at[j], denom_sp.at[dst_v.at[j]], dsem,
                         add=True)

        @pl.when(j >= 16)
        def _lagged_drain():
            pltpu.make_async_copy(ee_v.at[0], denom_sp.at[dst_v.at[0]],
                                  dsem).wait()
        return 0

    lax.fori_loop(0, NRW, row, 0)

    def drain(j, _):
        pltpu.make_async_copy(ee_v.at[0], denom_sp.at[dst_v.at[0]],
                              dsem).wait()
        return 0

    lax.fori_loop(0, 16, drain, 0)
    plsc.subcore_barrier()
    pltpu.sync_copy(denom_sp, denom_v)

    def row2(j, _):
        for k in range(CW // 16):
            dstv = dst_v[j, pl.ds(k * 16, 16)]
            dg = plsc.load_gather(denom_v, [dstv])
            a = ee_v[j, pl.ds(k * 16, 16)] / (dg + 1e-9) * 0.5
            ee_v[j, pl.ds(k * 16, 16)] = a
        return 0

    lax.fori_loop(0, NRW, row2, 0)
    pltpu.sync_copy(ee_v, alpha_out.at[c, s])


_ka = functools.partial(
    pl.kernel,
    out_type=jax.ShapeDtypeStruct((NC, NS, NRW, CW), jnp.float32),
    mesh=_MESH,
    compiler_params=_SC_PARAMS,
    scratch_types=[
        pltpu.VMEM_SHARED((N,), jnp.float32),
        pltpu.VMEM((N,), jnp.float32),
        pltpu.VMEM((N,), jnp.float32),
        pltpu.VMEM((NRW, CW), jnp.int32),
        pltpu.VMEM((NRW, CW), jnp.int32),
        pltpu.VMEM((NRW, CW), jnp.float32),
        pltpu.VMEM((N,), jnp.float32),
        pltpu.SemaphoreType.DMA,
    ],
)(_ka_body)


_NB = 4
_NQ = 4
_NH = NRW // _NQ


def _kb_body(tbl4, srcp, dstp, alphap, outh,
             accum_sp, src_v, dst_v, alpha_v,
             b0, b1, b2, b3, gs0, gs1, gs2, gs3, ss0, ss1, ss2, ss3):
    c = lax.axis_index("c")
    s = lax.axis_index("s")
    bufs = [b0, b1, b2, b3]
    gsems = [gs0, gs1, gs2, gs3]
    ssems = [ss0, ss1, ss2, ss3]

    def zb(i, _):
        for u in range(H // 16):
            b0[i, pl.ds(u * 16, 16)] = jnp.zeros((16,), jnp.float32)
        return 0
    lax.fori_loop(0, CW, zb, 0)
    row0 = s * RPT

    def stage_rows(write_out):
        def one(lo, sz):
            if write_out:
                pltpu.sync_copy(
                    accum_sp.at[pl.ds(row0 + lo, sz)],
                    outh.at[pl.ds(row0 + lo, sz), pl.ds(c * H, H)])
            else:
                pltpu.sync_copy(b0.at[pl.ds(0, sz)],
                                accum_sp.at[pl.ds(row0 + lo, sz)])
        for i in range(8):
            one(i * CW, CW)

        @pl.when(s < NS - 1)
        def _full():
            one(8 * CW, CW)
            one(9 * CW, RPT - 9 * CW)

        @pl.when(s == NS - 1)
        def _tail():
            one(8 * CW, _TAIL - 8 * CW)

    stage_rows(False)
    plsc.subcore_barrier()

    def seg_loop(seg, _):
        m = seg // _NQ
        hh = seg % _NQ
        off = 2 * N * m + c * N
        q0 = pl.multiple_of(hh * _NH, 8)
        pltpu.sync_copy(srcp.at[m, s, pl.ds(q0, _NH)], src_v)
        pltpu.sync_copy(dstp.at[m, s, pl.ds(q0, _NH)], dst_v)
        pltpu.sync_copy(alphap.at[m, s, pl.ds(q0, _NH)], alpha_v)

        def gather_issue(jj, b):
            for u in range(CW // 16):
                src_v[jj, pl.ds(u * 16, 16)] = (
                    src_v[jj, pl.ds(u * 16, 16)] + off)
            pltpu.async_copy(tbl4.at[src_v.at[jj]], bufs[b], gsems[b])

        def gather_wait(jj, b):
            pltpu.make_async_copy(
                tbl4.at[src_v.at[jj]], bufs[b], gsems[b]).wait()

        def do_scale(jj, b):
            rv = jnp.full((16,), jj, jnp.int32)

            def body(k, _):
                ab = plsc.load_gather(
                    alpha_v, [rv, jnp.full((16,), k, jnp.int32)])
                for u in range(H // 16):
                    bufs[b][k, pl.ds(u * 16, 16)] = (
                        bufs[b][k, pl.ds(u * 16, 16)] * ab)
                return 0
            lax.fori_loop(0, CW, body, 0, unroll=8)

        def scatter_issue(jj, b):
            pltpu.async_copy(bufs[b], accum_sp.at[dst_v.at[jj]],
                             ssems[b], add=True)

        def scatter_wait(jj, b):
            pltpu.make_async_copy(bufs[b], accum_sp.at[dst_v.at[jj]],
                                  ssems[b]).wait()

        for b in range(_NB):
            gather_issue(b, b)

        def outer(g, _):
            for b in range(_NB):
                jj = _NB * g + b
                gather_wait(jj, b)
                do_scale(jj, b)
                scatter_issue(jj, b)
                tt = jj + 2
                bb = (b + 2) % _NB

                @pl.when(jnp.logical_and(tt >= _NB, tt < _NH))
                def _recycle(tt=tt, bb=bb):
                    scatter_wait(tt - _NB, bb)
                    gather_issue(tt, bb)
            return 0

        lax.fori_loop(0, _NH // _NB, outer, 0)
        for b in range(_NB):
            scatter_wait(_NH - _NB + b, b)
        return 0

    lax.fori_loop(0, 2 * _NQ, seg_loop, 0)

    plsc.subcore_barrier()
    stage_rows(True)


_kb = functools.partial(
    pl.kernel,
    out_type=jax.ShapeDtypeStruct((N, DOUT), jnp.float32),
    mesh=_MESH,
    compiler_params=_SC_PARAMS,
    scratch_types=(
        [pltpu.VMEM_SHARED((N, H), jnp.float32),
         pltpu.VMEM((_NH, CW), jnp.int32),
         pltpu.VMEM((_NH, CW), jnp.int32),
         pltpu.VMEM((_NH, CW), jnp.float32)]
        + [pltpu.VMEM((CW, H), jnp.float32)] * _NB
        + [pltpu.SemaphoreType.DMA] * (2 * _NB)
    ),
)(_kb_body)


def kernel(h, edge_index0, edge_index1, W0, al0, ar0, W1, al1, ar1):
    Ws = jnp.stack([W0, W1])
    As = jnp.stack([jnp.stack([al0, ar0]), jnp.stack([al1, ar1])])
    Wh = jnp.stack([W0[:, :H], W0[:, H:], W1[:, :H], W1[:, H:]])
    sc = _tc_scores(h, Ws, As)
    tbl4 = _tc_tbl(h, Wh)

    pad = (jnp.arange(PADN, dtype=jnp.int32) * 97) % N

    def prep(ei):
        sfull = jnp.concatenate([ei[0], pad]).reshape(NS, NRW, CW)
        dfull = jnp.concatenate([ei[1], pad]).reshape(NS, NRW, CW)
        return sfull, dfull

    s0, d0 = prep(edge_index0)
    s1, d1 = prep(edge_index1)
    srcp = jnp.stack([s0, s1])
    dstp = jnp.stack([d0, d1])

    alpha = _ka(sc[:, 0], sc[:, 1], sc[:, 2], sc[:, 3], srcp, dstp)
    return _kb(tbl4, srcp, dstp, alpha)
